# Initial kernel scaffold; baseline (speedup 1.0000x reference)
import jax
import jax.numpy as jnp
from jax.experimental import pallas as pl


def kernel(p, params):
    raise NotImplementedError("write your pallas kernel here")


# speedup vs baseline: 1.4399x; 1.4399x over previous
import functools

import jax
import jax.numpy as jnp
from jax.experimental import pallas as pl
from jax.experimental.pallas import tpu as pltpu

_F3 = 12
_KF = 16
_EPS_BN = 1e-5
_EPS_D = 1e-9


def _dist2(qx, qy, qz, px, py, pz):
    dx = qx - px
    dy = qy - py
    dz = qz - pz
    return dx * dx + dy * dy + dz * dz


def _select_min(d2, iota, n):
    m = jnp.min(d2, axis=1, keepdims=True)
    first = jnp.min(jnp.where(d2 <= m, iota, n), axis=1, keepdims=True)
    oh = iota == first
    return oh, jnp.where(oh, jnp.float32(jnp.inf), d2)


def _masked_pick(oh, row):
    return jnp.sum(jnp.where(oh, row, jnp.float32(0.0)), axis=1, keepdims=True)


def _bdot(a, b):
    return jnp.dot(a.astype(jnp.bfloat16), b.astype(jnp.bfloat16),
                   preferred_element_type=jnp.float32)


def _t(x):
    return x.astype(jnp.bfloat16).astype(jnp.float32)


def _geo(dist, relx, rely, relz, rf):
    tx, ty, tz, trf = relx, rely, relz, rf
    cols = [dist]
    for f in range(_F3):
        cols.append(tx * trf[:, 3 * f:3 * f + 1]
                    + ty * trf[:, 3 * f + 1:3 * f + 2]
                    + tz * trf[:, 3 * f + 2:3 * f + 3])
    return jnp.concatenate(cols, axis=1)


def _cov_kernel(pT_ref, q_ref, nb_ref):
    pT = pT_ref[0]
    q = q_ref[0]
    px, py, pz = pT[0:1, :], pT[1:2, :], pT[2:3, :]
    pmx = jnp.mean(px, axis=1, keepdims=True)
    pmy = jnp.mean(py, axis=1, keepdims=True)
    pmz = jnp.mean(pz, axis=1, keepdims=True)
    qx = jnp.concatenate([q[:, 0:1], pmx], axis=0)
    qy = jnp.concatenate([q[:, 1:2], pmy], axis=0)
    qz = jnp.concatenate([q[:, 2:3], pmz], axis=0)
    d2 = _dist2(qx, qy, qz, px, py, pz)
    n = d2.shape[1]
    iota = jax.lax.broadcasted_iota(jnp.int32, d2.shape, 1)
    xs, ys, zs = [], [], []
    for _ in range(_KF):
        oh, d2 = _select_min(d2, iota, n)
        xs.append(_masked_pick(oh, px))
        ys.append(_masked_pick(oh, py))
        zs.append(_masked_pick(oh, pz))
    nb_ref[0] = jnp.concatenate(xs + ys + zs, axis=1)


def _feat0_kernel(p_ref, r3t_ref, wxr_ref, bxr_ref, y0_ref):
    p = p_ref[0]
    r3t = r3t_ref[0]
    pm = jnp.mean(p, axis=0, keepdims=True)
    d = p - pm
    pg = (d[:, 0:1] * r3t[0:1, :] + d[:, 1:2] * r3t[1:2, :]
          + d[:, 2:3] * r3t[2:3, :])
    y0_ref[0] = _bdot(pg, wxr_ref[:, :]) + bxr_ref[:, :]


def _rsconv_kernel(y_ref, pT_ref, pd_ref, rf_ref, g_ref, be_ref,
                   wm1_ref, bm1_ref, wm2_ref, bm2_ref, wr_ref,
                   out_ref, d2_ref, agg_ref, *, k, n_src, m_dst, rows_per_b):
    b = pl.program_id(0)
    y = y_ref[:, :]
    m0 = jnp.mean(y, axis=0, keepdims=True)
    v0 = jnp.mean((y - m0) ** 2, axis=0, keepdims=True)
    yb = y_ref[pl.ds(b * rows_per_b, rows_per_b), :]
    h = jax.nn.relu((yb - m0) / jnp.sqrt(v0 + _EPS_BN)
                    * g_ref[:, :] + be_ref[:, :])

    pT = pT_ref[0]
    px, py, pz = pT[0:1, :], pT[1:2, :], pT[2:3, :]
    pd = pd_ref[0]
    qx, qy, qz = pd[:, 0:1], pd[:, 1:2], pd[:, 2:3]
    d2_ref[:, :] = _dist2(qx, qy, qz, px, py, pz)
    agg_ref[:, :] = jnp.full_like(agg_ref, -jnp.inf)
    iota = jax.lax.broadcasted_iota(jnp.int32, (m_dst, n_src), 1)
    rf = rf_ref[0]
    wm1 = wm1_ref[:, :]
    bm1 = bm1_ref[:, :]
    wm2 = wm2_ref[:, :]
    bm2 = bm2_ref[:, :]

    def body(_, carry):
        oh, d2n = _select_min(d2_ref[:, :], iota, n_src)
        d2_ref[:, :] = d2n
        ohf = oh.astype(jnp.float32)
        relx = _masked_pick(oh, px) - qx
        rely = _masked_pick(oh, py) - qy
        relz = _masked_pick(oh, pz) - qz
        dist = jnp.sqrt(relx * relx + rely * rely + relz * relz + _EPS_D)
        geo = _geo(dist, relx, rely, relz, rf)
        u = _bdot(geo, wm1) + bm1
        w = _bdot(jax.nn.relu(u), wm2) + bm2
        nb_h = jnp.dot(ohf, h, preferred_element_type=jnp.float32,
                       precision=jax.lax.Precision.HIGHEST)
        agg_ref[:, :] = jnp.maximum(agg_ref[:, :], w * nb_h)
        return carry

    jax.lax.fori_loop(0, k, body, 0)
    out_ref[0] = _bdot(agg_ref[:, :], wr_ref[:, :])


def _head_kernel(y2_ref, p_ref, p2_ref, r3f_ref, g2_ref, be2_ref,
                 wm1_ref, bm1_ref, wm2_ref, bm2_ref, wr_ref,
                 g3_ref, be3_ref, wc1_ref, gc1_ref, bec1_ref,
                 wc2_ref, gc2_ref, bec2_ref, wc3_ref, bc3_ref, out_ref):
    y2 = y2_ref[:, :]
    m = jnp.mean(y2, axis=0, keepdims=True)
    v = jnp.mean((y2 - m) ** 2, axis=0, keepdims=True)
    h2 = jax.nn.relu((y2 - m) / jnp.sqrt(v + _EPS_BN)
                     * g2_ref[:, :] + be2_ref[:, :])

    wm1 = wm1_ref[:, :]
    bm1 = bm1_ref[:, :]
    wm2 = wm2_ref[:, :]
    bm2 = bm2_ref[:, :]
    aggs = []
    for b in range(16):
        pb = p_ref[pl.ds(b * 1024, 1024), :]
        pm = jnp.mean(pb, axis=0, keepdims=True)
        p2b = p2_ref[pl.ds(b * 128, 128), :]
        rel = p2b - pm
        dist = jnp.sqrt(jnp.sum(rel * rel, axis=1, keepdims=True) + _EPS_D)
        rf = r3f_ref[b:b + 1, :]
        geo = _geo(dist, rel[:, 0:1], rel[:, 1:2], rel[:, 2:3], rf)
        u = _bdot(geo, wm1) + bm1
        w = _bdot(jax.nn.relu(u), wm2) + bm2
        h2b = h2[b * 128:(b + 1) * 128, :]
        aggs.append(jnp.max(w * h2b, axis=0, keepdims=True))
    agg = jnp.concatenate(aggs, axis=0)
    y3 = _bdot(agg, wr_ref[:, :])

    def bn_rows(x, g, be):
        mu = jnp.mean(x, axis=0, keepdims=True)
        va = jnp.mean((x - mu) ** 2, axis=0, keepdims=True)
        return (x - mu) / jnp.sqrt(va + _EPS_BN) * g + be

    h3 = jax.nn.relu(bn_rows(y3, g3_ref[:, :], be3_ref[:, :]))
    x = bn_rows(_bdot(h3, wc1_ref[:, :]), gc1_ref[:, :], bec1_ref[:, :])
    x = bn_rows(_bdot(x, wc2_ref[:, :]), gc2_ref[:, :], bec2_ref[:, :])
    out_ref[:, :] = _bdot(x, wc3_ref[:, :]) + bc3_ref[:, :]


def _full(shape):
    return pl.BlockSpec(shape, lambda b: tuple(0 for _ in shape))


def _perb(shape):
    return pl.BlockSpec((1,) + shape, lambda b: (b,) + tuple(0 for _ in shape))


def _row(a):
    return a.reshape(1, -1)


def kernel(p, params):
    P = params
    B, N = p.shape[0], p.shape[1]
    p1 = p[:, ::2]
    p2 = p[:, ::8]
    pT = jnp.swapaxes(p, 1, 2)
    p1T = jnp.swapaxes(p1, 1, 2)
    p2T = jnp.swapaxes(p2, 1, 2)
    q = jnp.concatenate([p1, p2], axis=1)

    nb48 = pl.pallas_call(
        _cov_kernel,
        grid=(B,),
        in_specs=[_perb((3, N)), _perb((640, 3))],
        out_specs=_perb((641, 48)),
        out_shape=jax.ShapeDtypeStruct((B, 641, 48), jnp.float32),
    )(pT, q)
    nb = jnp.swapaxes(nb48.reshape(B, 641, 3, _KF), 2, 3)

    signs = jnp.array([[1., 1., 1.], [-1., -1., 1.],
                       [-1., 1., -1.], [1., -1., -1.]], dtype=jnp.float32)

    def frames_from_nb(nbl):
        c = nbl - nbl.mean(axis=2, keepdims=True)
        cov = jnp.einsum('bqki,bqkj->bqij', c, c) / _KF
        _, V = jnp.linalg.eigh(cov)
        Vt = jnp.swapaxes(V, -1, -2)
        R = signs[None, None, :, :, None] * Vt[:, :, None, :, :]
        return R.reshape(R.shape[0], R.shape[1], _F3, 3)

    Rf1 = frames_from_nb(nb[:, :512])
    Rf2 = frames_from_nb(nb[:, 512:640])
    Rf3 = frames_from_nb(nb[:, 640:641])
    R1f = Rf1.reshape(B, 512, 36)
    R2f = Rf2.reshape(B, 128, 36)
    R3 = Rf3[:, 0]
    R3f = R3.reshape(B, 36)
    R3T = jnp.swapaxes(R3, 1, 2)

    y0 = pl.pallas_call(
        _feat0_kernel,
        grid=(B,),
        in_specs=[_perb((N, 3)), _perb((3, 12)), _full((12, 32)),
                  _full((1, 32))],
        out_specs=_perb((N, 32)),
        out_shape=jax.ShapeDtypeStruct((B, N, 32), jnp.float32),
    )(p, R3T, P['Wxr'], _row(P['bxr']))

    def rsconv(y2d, pTsrc, pdst, rflat, g, be, wm1, bm1, wm2, bm2, wr,
               k, n_src, m_dst, cin, hdim, cout):
        f = functools.partial(_rsconv_kernel, k=k, n_src=n_src, m_dst=m_dst,
                              rows_per_b=n_src)
        return pl.pallas_call(
            f,
            grid=(B,),
            in_specs=[_full((B * n_src, cin)), _perb((3, n_src)),
                      _perb((m_dst, 3)), _perb((m_dst, 36)),
                      _full((1, cin)), _full((1, cin)),
                      _full((13, hdim)), _full((1, hdim)),
                      _full((hdim, cin)), _full((1, cin)),
                      _full((cin, cout))],
            out_specs=_perb((m_dst, cout)),
            out_shape=jax.ShapeDtypeStruct((B, m_dst, cout), jnp.float32),
            scratch_shapes=[pltpu.VMEM((m_dst, n_src), jnp.float32),
                            pltpu.VMEM((m_dst, cin), jnp.float32)],
        )(y2d, pTsrc, pdst, rflat, _row(g), _row(be),
          wm1, _row(bm1), wm2, _row(bm2), wr)

    y1 = rsconv(y0.reshape(B * N, 32), pT, p1, R1f, P['gxr'], P['bexr'],
                P['W1m1'], P['b1m1'], P['W1m2'], P['b1m2'], P['W1r'],
                48, N, 512, 32, 64, 128)
    y2 = rsconv(y1.reshape(B * 512, 128), p1T, p2, R2f, P['g1'], P['be1'],
                P['W2m1'], P['b2m1'], P['W2m2'], P['b2m2'], P['W2r'],
                64, 512, 128, 128, 64, 512)

    out = pl.pallas_call(
        _head_kernel,
        grid=(1,),
        in_specs=[_full((B * 128, 512)), _full((B * N, 3)),
                  _full((B * 128, 3)), _full((B, 36)),
                  _full((1, 512)), _full((1, 512)),
                  _full((13, 64)), _full((1, 64)),
                  _full((64, 512)), _full((1, 512)), _full((512, 1024)),
                  _full((1, 1024)), _full((1, 1024)),
                  _full((1024, 512)), _full((1, 512)), _full((1, 512)),
                  _full((512, 256)), _full((1, 256)), _full((1, 256)),
                  _full((256, 40)), _full((1, 40))],
        out_specs=_full((B, 40)),
        out_shape=jax.ShapeDtypeStruct((B, 40), jnp.float32),
    )(y2.reshape(B * 128, 512), p.reshape(B * N, 3), p2.reshape(B * 128, 3),
      R3f, _row(P['g2']), _row(P['be2']),
      P['W3m1'], _row(P['b3m1']), P['W3m2'], _row(P['b3m2']), P['W3r'],
      _row(P['g3']), _row(P['be3']),
      P['Wc1'], _row(P['gc1']), _row(P['bec1']),
      P['Wc2'], _row(P['gc2']), _row(P['bec2']),
      P['Wc3'], _row(P['bc3']))
    return out
